# baseline (speedup 1.0000x reference)
import jax
import jax.numpy as jnp
from jax.experimental import pallas as pl
from jax.experimental.pallas import tpu as pltpu

D = 512
H = 8
N = 16384
BLK = 1024
NB = N // BLK
OMPAD = 640
NEG = -1e30
F32 = jnp.float32
BF = jnp.bfloat16

_SELU_SCALE = 1.0507009873554805
_SELU_ALPHA = 1.6732632423543772


def _selu(x):
    neg = _SELU_ALPHA * (jnp.exp(jnp.minimum(x, 0.0)) - 1.0)
    return _SELU_SCALE * jnp.where(x > 0, x, neg)


def _omic_kernel(xom_ref, w1_ref, b1_ref, w2_ref, b2_ref, out_ref):
    for i in range(6):
        h = jnp.dot(xom_ref[i : i + 1, :], w1_ref[i],
                    preferred_element_type=F32)
        h = _selu(h + b1_ref[i : i + 1, :])
        h = jnp.dot(h.astype(BF), w2_ref[i], preferred_element_type=F32)
        h = _selu(h + b2_ref[i : i + 1, :])
        out_ref[i : i + 1, :] = h
    out_ref[6:8, :] = jnp.zeros((2, D), F32)


def _dot(a, b):
    return jnp.dot(a, b, preferred_element_type=F32)


def _dot_nt(a, b):
    return jax.lax.dot_general(a, b, (((1,), (1,)), ((), ())),
                               preferred_element_type=F32)


def _dot2(a, w_hi_ref, w_lo_ref):
    return _dot(a, w_hi_ref[...]) + _dot(a, w_lo_ref[...])


def _path_attn(hb, wqkh_ref, wqkl_ref, sbias_ref, gt_ref, vwfh_ref,
               vwfl_ref, hbias_ref):
    S = _dot2(hb, wqkh_ref, wqkl_ref) + sbias_ref[...]
    M = jnp.max(S, axis=1, keepdims=True)
    E = jnp.exp(S - M)
    Dn = _dot(E, gt_ref[...])
    A = (E / Dn).astype(BF)
    return jnp.maximum(_dot2(A, vwfh_ref, vwfl_ref) + hbias_ref[...], 0.0)


def _stream_omic(qwh_ref, qwl_ref, hb, m_scr, l_scr, u_scr):
    S = _dot_nt(qwh_ref[...], hb) + _dot_nt(qwl_ref[...], hb)
    m_old = m_scr[:, 0:1]
    m_new = jnp.maximum(m_old, jnp.max(S, axis=1, keepdims=True))
    alpha = jnp.exp(m_old - m_new)
    P = jnp.exp(S - m_new)
    l_scr[:, 0:1] = l_scr[:, 0:1] * alpha + jnp.sum(P, axis=1, keepdims=True)
    u_scr[...] = u_scr[...] * alpha + _dot(P.astype(BF), hb)
    m_scr[:, 0:1] = m_new


def _finish_omic(m_scr, l_scr, u_scr, wv_ref, bv_ref, wo_ref, bo_ref,
                 wf_ref, bf_ref):
    an = u_scr[...] / l_scr[:, 0:1]
    z = _dot(an, wv_ref[...].astype(F32))
    o_om = jnp.concatenate(
        [z[h * 8 : (h + 1) * 8, h * 64 : (h + 1) * 64] for h in range(H)],
        axis=1) + bv_ref[...]
    attn = _dot(o_om.astype(BF), wo_ref[...]) + bo_ref[...]
    return jnp.maximum(_dot(attn.astype(BF), wf_ref[...]) + bf_ref[...],
                       0.0)


def _passA_kernel(x_ref, wsiW_ref, wsib_ref,
                  wqk0h_ref, wqk0l_ref, sb0_ref, gt_ref, vwf0h_ref,
                  vwf0l_ref, hb0_ref,
                  qw1h_ref, qw1l_ref, wv1_ref, bv1_ref, wo1_ref, bo1_ref,
                  wf1_ref, bf1_ref,
                  h1_ref, hom1_ref,
                  m_scr, l_scr, u_scr):
    i = pl.program_id(0)

    @pl.when(i == 0)
    def _init():
        m_scr[...] = jnp.full(m_scr.shape, NEG, F32)
        l_scr[...] = jnp.zeros(l_scr.shape, F32)
        u_scr[...] = jnp.zeros(u_scr.shape, F32)

    h0 = jnp.maximum(_dot(x_ref[...], wsiW_ref[...]) + wsib_ref[...], 0.0)
    h1 = _path_attn(h0.astype(BF), wqk0h_ref, wqk0l_ref, sb0_ref, gt_ref,
                    vwf0h_ref, vwf0l_ref, hb0_ref)
    h1b = h1.astype(BF)
    h1_ref[...] = h1b
    _stream_omic(qw1h_ref, qw1l_ref, h1b, m_scr, l_scr, u_scr)

    @pl.when(i == NB - 1)
    def _fin():
        hom1_ref[...] = _finish_omic(m_scr, l_scr, u_scr, wv1_ref, bv1_ref,
                                     wo1_ref, bo1_ref, wf1_ref, bf1_ref)


def _passB_kernel(h1_ref,
                  wqk2h_ref, wqk2l_ref, sb2_ref, gt_ref, vwf2h_ref,
                  vwf2l_ref, hb2_ref,
                  qw3h_ref, qw3l_ref, wv3_ref, bv3_ref, wo3_ref, bo3_ref,
                  wf3_ref, bf3_ref,
                  sav0_ref, saw0_ref, sav1_ref, saw1_ref,
                  clsW_ref, clsb_ref,
                  out_ref,
                  m_scr, l_scr, u_scr, pm_scr, pl_scr, pacc_scr):
    i = pl.program_id(0)

    @pl.when(i == 0)
    def _init():
        m_scr[...] = jnp.full(m_scr.shape, NEG, F32)
        l_scr[...] = jnp.zeros(l_scr.shape, F32)
        u_scr[...] = jnp.zeros(u_scr.shape, F32)
        pm_scr[...] = jnp.full(pm_scr.shape, NEG, F32)
        pl_scr[...] = jnp.zeros(pl_scr.shape, F32)
        pacc_scr[...] = jnp.zeros(pacc_scr.shape, F32)

    h2 = _path_attn(h1_ref[...], wqk2h_ref, wqk2l_ref, sb2_ref, gt_ref,
                    vwf2h_ref, vwf2l_ref, hb2_ref)
    h2b = h2.astype(BF)

    t = jnp.tanh(_dot(h2b, sav0_ref[...]))
    sT = _dot_nt(saw0_ref[...], t.astype(BF))
    m_old = pm_scr[0:1, 0:1]
    m_new = jnp.maximum(m_old, jnp.max(sT, axis=1, keepdims=True))
    alpha = jnp.exp(m_old - m_new)
    p = jnp.exp(sT - m_new)
    pl_scr[0:1, 0:1] = pl_scr[0:1, 0:1] * alpha + jnp.sum(
        p, axis=1, keepdims=True)
    pacc_scr[0:1, :] = pacc_scr[0:1, :] * alpha + _dot(p.astype(BF), h2b)
    pm_scr[0:1, 0:1] = m_new

    _stream_omic(qw3h_ref, qw3l_ref, h2b, m_scr, l_scr, u_scr)

    @pl.when(i == NB - 1)
    def _fin():
        hom_f = _finish_omic(m_scr, l_scr, u_scr, wv3_ref, bv3_ref,
                             wo3_ref, bo3_ref, wf3_ref, bf3_ref)
        t2 = jnp.tanh(_dot(hom_f.astype(BF), sav1_ref[...]))
        s2 = _dot_nt(saw1_ref[...], t2.astype(BF))
        mask = jax.lax.broadcasted_iota(jnp.int32, s2.shape, 1) >= 6
        s2 = jnp.where(mask, NEG, s2)
        mx = jnp.max(s2, axis=1, keepdims=True)
        e = jnp.exp(s2 - mx)
        a = (e / jnp.sum(e, axis=1, keepdims=True)).astype(BF)
        ho = _dot(a, hom_f.astype(BF))
        hp = pacc_scr[0:1, :] / pl_scr[0:1, 0:1]
        hcat = jnp.concatenate([hp, ho], axis=1)
        logits = _dot(hcat.astype(BF), clsW_ref[...]) + clsb_ref[...]
        out_ref[...] = jnp.broadcast_to(logits, out_ref.shape)


def _full(shape):
    return pl.BlockSpec(shape, lambda i: (0,) * len(shape))


def _hdot(a, b):
    return jnp.matmul(a, b, precision=jax.lax.Precision.HIGHEST)


def _split(w):
    ui = jax.lax.bitcast_convert_type(w, jnp.uint32)
    hi32 = jax.lax.bitcast_convert_type(ui & jnp.uint32(0xFFFF0000), F32)
    return hi32.astype(BF), (w - hi32).astype(BF)


def _bfr(w):
    ui = jax.lax.bitcast_convert_type(w, jnp.uint32)
    r = (ui + jnp.uint32(0x7FFF) + ((ui >> 16) & jnp.uint32(1))) \
        & jnp.uint32(0xFFFF0000)
    return jax.lax.bitcast_convert_type(r, F32)


def _fold_path_layer(hom6, Wq, bq, Wk, bk, Wv, bv, Wo, bo, Wf, bf):
    k = _bfr(jnp.matmul(hom6, Wk) + bk)
    v = _bfr(jnp.matmul(hom6, Wv) + bv)
    i8 = jnp.eye(8, dtype=F32)
    kh = jnp.pad(k.reshape(6, 8, 64).transpose(1, 2, 0),
                 ((0, 0), (0, 0), (0, 2)))
    kbd = jnp.einsum('hdj,hH->hdHj', kh, i8).reshape(D, 64)
    wqk = _hdot(_bfr(Wq), kbd) * 0.125
    lane = jnp.arange(64) % 8
    sbias = ((_hdot(bq, kbd)) * 0.125 + jnp.where(lane >= 6, NEG, 0.0)
             ).reshape(1, 64)
    vh = jnp.pad(v.reshape(6, 8, 64).transpose(1, 0, 2),
                 ((0, 0), (0, 2), (0, 0)))
    vbd = jnp.einsum('hjd,hH->hjHd', vh, i8).reshape(64, D)
    vwf = _hdot(_hdot(vbd, _bfr(Wo)), _bfr(Wf))
    hbias = (_hdot(bo, _bfr(Wf)) + bf).reshape(1, D)
    return _split(wqk), sbias, _split(vwf), hbias


def _fold_omic_queries(hom8, Wq, bq, Wk):
    q = _bfr(jnp.matmul(hom8, Wq) + bq)
    qh = q.reshape(8, 8, 64).transpose(1, 0, 2)
    i8 = jnp.eye(8, dtype=F32)
    qbd = jnp.einsum('hqd,hH->hqHd', qh, i8).reshape(64, D)
    return _split(_hdot(qbd, _bfr(Wk).T) * 0.125)


def kernel(x_path, x_omic1, x_omic2, x_omic3, x_omic4, x_omic5, x_omic6,
           wsi_W, wsi_b,
           sig_W1_1, sig_W1_2, sig_W1_3, sig_W1_4, sig_W1_5, sig_W1_6,
           sig_b1, sig_W2, sig_b2,
           coa_Wq, coa_bq, coa_Wk, coa_bk, coa_Wv, coa_bv, coa_Wo, coa_bo,
           coa_Wf, coa_bf,
           sa_V, sa_w, cls_W, cls_b):
    xb = x_path[0].astype(BF)

    omics = [x_omic1, x_omic2, x_omic3, x_omic4, x_omic5, x_omic6]
    w1s = [sig_W1_1, sig_W1_2, sig_W1_3, sig_W1_4, sig_W1_5, sig_W1_6]
    xom = jnp.stack([jnp.pad(o, (0, OMPAD - o.shape[0])) for o in omics])
    w1 = jnp.stack([jnp.pad(w, ((0, OMPAD - w.shape[0]), (0, 0)))
                    for w in w1s])
    h_omic0 = pl.pallas_call(
        _omic_kernel,
        out_shape=jax.ShapeDtypeStruct((8, D), F32),
    )(xom.astype(BF), w1.astype(BF), sig_b1, sig_W2.astype(BF), sig_b2)

    gt = (jnp.arange(64)[:, None] // 8 ==
          jnp.arange(64)[None, :] // 8).astype(F32)
    brow = [b.reshape(4, 1, D) for b in (coa_bv, coa_bo, coa_bf)]

    (wqk0h, wqk0l), sb0, (vwf0h, vwf0l), hb0 = _fold_path_layer(
        h_omic0[:6], coa_Wq[0], coa_bq[0], coa_Wk[0], coa_bk[0],
        coa_Wv[0], coa_bv[0], coa_Wo[0], coa_bo[0], coa_Wf[0], coa_bf[0])
    qw1h, qw1l = _fold_omic_queries(h_omic0, coa_Wq[1], coa_bq[1],
                                    coa_Wk[1])

    wspec = _full((D, D))
    bspec = _full((1, D))

    h1, h_omic1 = pl.pallas_call(
        _passA_kernel,
        grid=(NB,),
        in_specs=[
            pl.BlockSpec((BLK, 1024), lambda i: (i, 0)),
            _full((1024, D)), bspec,
            _full((D, 64)), _full((D, 64)), _full((1, 64)),
            _full((64, 64)),
            _full((64, D)), _full((64, D)), bspec,
            _full((64, D)), _full((64, D)),
            wspec, bspec, wspec, bspec, wspec, bspec,
        ],
        out_specs=[
            pl.BlockSpec((BLK, D), lambda i: (i, 0)),
            _full((8, D)),
        ],
        out_shape=[
            jax.ShapeDtypeStruct((N, D), BF),
            jax.ShapeDtypeStruct((8, D), F32),
        ],
        scratch_shapes=[
            pltpu.VMEM((64, 128), F32), pltpu.VMEM((64, 128), F32),
            pltpu.VMEM((64, D), F32),
        ],
    )(xb, wsi_W.astype(BF), wsi_b.reshape(1, D),
      wqk0h, wqk0l, sb0, gt, vwf0h, vwf0l, hb0,
      qw1h, qw1l, coa_Wv[1].astype(BF), brow[0][1],
      coa_Wo[1].astype(BF), brow[1][1], coa_Wf[1].astype(BF),
      brow[2][1])

    (wqk2h, wqk2l), sb2, (vwf2h, vwf2l), hb2 = _fold_path_layer(
        h_omic1[:6], coa_Wq[2], coa_bq[2], coa_Wk[2], coa_bk[2],
        coa_Wv[2], coa_bv[2], coa_Wo[2], coa_bo[2], coa_Wf[2], coa_bf[2])
    qw3h, qw3l = _fold_omic_queries(h_omic1, coa_Wq[3], coa_bq[3],
                                    coa_Wk[3])

    clsW_pad = jnp.pad(cls_W, ((0, 0), (0, 128 - cls_W.shape[1])))
    clsb_pad = jnp.pad(cls_b, (0, 128 - cls_b.shape[0])).reshape(1, 128)

    out = pl.pallas_call(
        _passB_kernel,
        grid=(NB,),
        in_specs=[
            pl.BlockSpec((BLK, D), lambda i: (i, 0)),
            _full((D, 64)), _full((D, 64)), _full((1, 64)),
            _full((64, 64)),
            _full((64, D)), _full((64, D)), bspec,
            _full((64, D)), _full((64, D)),
            wspec, bspec, wspec, bspec, wspec, bspec,
            _full((D, 128)), _full((1, 128)), _full((D, 128)),
            _full((1, 128)),
            _full((2 * D, 128)), _full((1, 128)),
        ],
        out_specs=_full((8, 128)),
        out_shape=jax.ShapeDtypeStruct((8, 128), F32),
        scratch_shapes=[
            pltpu.VMEM((64, 128), F32), pltpu.VMEM((64, 128), F32),
            pltpu.VMEM((64, D), F32), pltpu.VMEM((1, 128), F32),
            pltpu.VMEM((1, 128), F32), pltpu.VMEM((1, D), F32),
        ],
    )(h1,
      wqk2h, wqk2l, sb2, gt, vwf2h, vwf2l, hb2,
      qw3h, qw3l, coa_Wv[3].astype(BF), brow[0][3],
      coa_Wo[3].astype(BF), brow[1][3], coa_Wf[3].astype(BF),
      brow[2][3],
      sa_V[0].astype(BF), sa_w[0:1].astype(BF), sa_V[1].astype(BF),
      sa_w[1:2].astype(BF),
      clsW_pad.astype(BF), clsb_pad)

    return out[0:1, 0:4]
